# Initial kernel scaffold; baseline (speedup 1.0000x reference)
import functools

import jax
import jax.numpy as jnp
from jax import lax
from jax.experimental import pallas as pl
from jax.experimental.pallas import tpu as pltpu
from jax.experimental.pallas import tpu_sc as plsc

B = 1024
K = 8192
D = 256
M = 32
M1 = 33

BQ = 64


def _topk_body(latT_ref, z_ref, idx_ref):
    latT = latT_ref[...]
    zb = z_ref[...]
    ln = jnp.sum(latT * latT, axis=0, keepdims=True)
    dot = lax.dot_general(zb, latT, (((1,), (0,)), ((), ())),
                          precision=lax.Precision.HIGHEST,
                          preferred_element_type=jnp.float32)
    s = ln - 2.0 * dot
    iota = lax.broadcasted_iota(jnp.int32, (BQ, K), 1)
    inf = jnp.float32(jnp.inf)
    for t in range(M1):
        mval = jnp.min(s, axis=1, keepdims=True)
        imin = jnp.min(jnp.where(s == mval, iota, K), axis=1, keepdims=True)
        if t > 0:
            idx_ref[:, t - 1:t] = imin
        s = jnp.where(iota == imin, inf, s)


def _topk_indices(latT, z):
    return pl.pallas_call(
        _topk_body,
        grid=(B // BQ,),
        in_specs=[
            pl.BlockSpec((D, K), lambda i: (0, 0)),
            pl.BlockSpec((BQ, D), lambda i: (i, 0)),
        ],
        out_specs=pl.BlockSpec((BQ, M), lambda i: (i, 0)),
        out_shape=jax.ShapeDtypeStruct((B, M), jnp.int32),
    )(latT, z)


NC = 2
NS = 16
NW = NC * NS
ROWS = B * M
B_PER_W = ROWS // NW
CHUNK = 128


def _gather_rows(table, idx_flat):
    mesh = plsc.VectorSubcoreMesh(core_axis_name="c", subcore_axis_name="s")

    @functools.partial(
        pl.kernel,
        mesh=mesh,
        out_type=jax.ShapeDtypeStruct((ROWS, D), jnp.float32),
        scratch_types=[
            pltpu.VMEM((CHUNK,), jnp.int32),
            pltpu.VMEM((CHUNK, D), jnp.float32),
            pltpu.SemaphoreType.DMA,
        ],
    )
    def k(table_hbm, idx_hbm, out_hbm, idx_v, rows_v, sem):
        wid = lax.axis_index("s") * NC + lax.axis_index("c")
        for c in range(B_PER_W // CHUNK):
            base = wid * B_PER_W + c * CHUNK
            pltpu.sync_copy(idx_hbm.at[pl.ds(base, CHUNK)], idx_v)
            pltpu.async_copy(table_hbm.at[idx_v], rows_v, sem).wait()
            pltpu.sync_copy(rows_v, out_hbm.at[pl.ds(base, CHUNK)])

    return k(table, idx_flat)


FQ = 8
NBLK = B // FQ


def _fit_body(sel_ref, zrep_ref, lin_ref, loss_ref):
    pid = pl.program_id(0)
    sel = sel_ref[...]
    X = sel - zrep_ref[...]
    G = lax.dot_general(X, X, (((1,), (1,)), ((), ())),
                        precision=lax.Precision.HIGHEST,
                        preferred_element_type=jnp.float32)
    A = jnp.stack([G[M * q:M * q + M, M * q:M * q + M] for q in range(FQ)],
                  axis=0)
    bvec = jnp.ones((FQ, M), jnp.float32)
    iota1 = lax.broadcasted_iota(jnp.int32, (FQ, M, 1), 1)
    for k in range(M):
        piv = A[:, k:k + 1, k:k + 1]
        prow = A[:, k:k + 1, :] / piv
        pb = bvec[:, k:k + 1] / piv[:, :, 0]
        fac = A[:, :, k:k + 1]
        maskrow = iota1 == k
        A = jnp.where(maskrow, prow, A - fac * prow)
        bvec = jnp.where(maskrow[:, :, 0], pb, bvec - fac[:, :, 0] * pb)
    w = bvec
    w = w / jnp.sum(w, axis=1, keepdims=True)
    w_tiled = jnp.tile(w, (1, FQ))
    qlane = lax.broadcasted_iota(jnp.int32, (FQ, FQ * M), 1) // M
    qrow = lax.broadcasted_iota(jnp.int32, (FQ, FQ * M), 0)
    w_sel = jnp.where(qlane == qrow, w_tiled, 0.0)
    lin_ref[...] = lax.dot_general(w_sel, sel, (((1,), (0,)), ((), ())),
                                   precision=lax.Precision.HIGHEST,
                                   preferred_element_type=jnp.float32)
    psum = jnp.sum(jnp.clip(-w, 0.0, 1000.0))

    @pl.when(pid == 0)
    def _():
        loss_ref[0, 0] = 0.0

    loss_ref[0, 0] += psum

    @pl.when(pid == NBLK - 1)
    def _():
        loss_ref[0, 0] = loss_ref[0, 0] * (1.0 / (B * M))


def _fit(sel_flat, zrep):
    return pl.pallas_call(
        _fit_body,
        grid=(NBLK,),
        in_specs=[
            pl.BlockSpec((FQ * M, D), lambda i: (i, 0)),
            pl.BlockSpec((FQ * M, D), lambda i: (i, 0)),
        ],
        out_specs=[
            pl.BlockSpec((FQ, D), lambda i: (i, 0)),
            pl.BlockSpec((1, 1), lambda i: (0, 0)),
        ],
        out_shape=[
            jax.ShapeDtypeStruct((B, D), jnp.float32),
            jax.ShapeDtypeStruct((1, 1), jnp.float32),
        ],
    )(sel_flat, zrep)


def kernel(gt_x, latents, z):
    del gt_x
    latT = latents.T
    idx = _topk_indices(latT, z)
    sel = _gather_rows(latents, idx.reshape(-1))
    zrep = jnp.repeat(z, M, axis=0)
    lin, loss = _fit(sel, zrep)
    return (loss.reshape(()), lin)


# speedup vs baseline: 1.8524x; 1.8524x over previous
import functools

import jax
import jax.numpy as jnp
from jax import lax
from jax.experimental import pallas as pl
from jax.experimental.pallas import tpu as pltpu
from jax.experimental.pallas import tpu_sc as plsc

B = 1024
K = 8192
D = 256
M = 32
M1 = 33

BQ = 64


def _topk_body(latT_ref, z_ref, idx_ref):
    latT = latT_ref[...]
    zb = z_ref[...]
    ln = jnp.sum(latT * latT, axis=0, keepdims=True)
    dot = lax.dot_general(zb, latT, (((1,), (0,)), ((), ())),
                          precision=lax.Precision.HIGHEST,
                          preferred_element_type=jnp.float32)
    s = ln - 2.0 * dot
    iota = lax.broadcasted_iota(jnp.int32, (BQ, K), 1)
    inf = jnp.float32(jnp.inf)
    for t in range(M1):
        mval = jnp.min(s, axis=1, keepdims=True)
        imin = jnp.min(jnp.where(s == mval, iota, K), axis=1, keepdims=True)
        if t > 0:
            idx_ref[:, t - 1:t] = imin
        s = jnp.where(iota == imin, inf, s)


def _topk_indices(latT, z):
    return pl.pallas_call(
        _topk_body,
        grid=(B // BQ,),
        in_specs=[
            pl.BlockSpec((D, K), lambda i: (0, 0)),
            pl.BlockSpec((BQ, D), lambda i: (i, 0)),
        ],
        out_specs=pl.BlockSpec((BQ, M), lambda i: (i, 0)),
        out_shape=jax.ShapeDtypeStruct((B, M), jnp.int32),
    )(latT, z)


NC = 2
NS = 16
NW = NC * NS
ROWS = B * M
B_PER_W = ROWS // NW
CHUNK = 128


def _gather_rows(table, idx_flat):
    mesh = plsc.VectorSubcoreMesh(core_axis_name="c", subcore_axis_name="s")

    @functools.partial(
        pl.kernel,
        mesh=mesh,
        out_type=jax.ShapeDtypeStruct((ROWS, D), jnp.float32),
        scratch_types=[
            pltpu.VMEM((CHUNK,), jnp.int32),
            pltpu.VMEM((CHUNK, D), jnp.float32),
            pltpu.SemaphoreType.DMA,
        ],
    )
    def k(table_hbm, idx_hbm, out_hbm, idx_v, rows_v, sem):
        wid = lax.axis_index("s") * NC + lax.axis_index("c")
        for c in range(B_PER_W // CHUNK):
            base = wid * B_PER_W + c * CHUNK
            pltpu.sync_copy(idx_hbm.at[pl.ds(base, CHUNK)], idx_v)
            pltpu.async_copy(table_hbm.at[idx_v], rows_v, sem).wait()
            pltpu.sync_copy(rows_v, out_hbm.at[pl.ds(base, CHUNK)])

    return k(table, idx_flat)


FQ = 8
NBLK = B // FQ


def _fit_body(wn_ref, sel_ref, lin_ref, loss_ref):
    pid = pl.program_id(0)
    sel = sel_ref[...]
    wn = wn_ref[...]
    w_tiled = jnp.tile(wn, (1, FQ))
    qlane = lax.broadcasted_iota(jnp.int32, (FQ, FQ * M), 1) // M
    qrow = lax.broadcasted_iota(jnp.int32, (FQ, FQ * M), 0)
    w_sel = jnp.where(qlane == qrow, w_tiled, 0.0)
    lin_ref[...] = lax.dot_general(w_sel, sel, (((1,), (0,)), ((), ())),
                                   precision=lax.Precision.HIGHEST,
                                   preferred_element_type=jnp.float32)
    psum = jnp.sum(jnp.clip(-wn, 0.0, 1000.0)).reshape(1, 1)

    @pl.when(pid == 0)
    def _():
        loss_ref[...] = jnp.zeros((1, 1), jnp.float32)

    loss_ref[...] += psum

    @pl.when(pid == NBLK - 1)
    def _():
        loss_ref[...] = loss_ref[...] * (1.0 / (B * M))


def _fit(wn, sel_flat):
    return pl.pallas_call(
        _fit_body,
        grid=(NBLK,),
        in_specs=[
            pl.BlockSpec((FQ, M), lambda i: (i, 0)),
            pl.BlockSpec((FQ * M, D), lambda i: (i, 0)),
        ],
        out_specs=[
            pl.BlockSpec((FQ, D), lambda i: (i, 0)),
            pl.BlockSpec((1, 1), lambda i: (0, 0)),
        ],
        out_shape=[
            jax.ShapeDtypeStruct((B, D), jnp.float32),
            jax.ShapeDtypeStruct((1, 1), jnp.float32),
        ],
    )(wn, sel_flat)


def kernel(gt_x, latents, z):
    del gt_x
    latT = latents.T
    idx = _topk_indices(latT, z)
    sel = _gather_rows(latents, idx.reshape(-1))
    lc = sel.reshape(B, M, D) - z[:, None, :]
    A = jnp.einsum('bmd,bnd->bmn', lc, lc)
    ainv = jnp.linalg.inv(A)
    ones = jnp.ones((B, M, 1), dtype=jnp.float32)
    w = jnp.matmul(ainv, ones)
    wn = w / jnp.sum(w, axis=1, keepdims=True)
    lin, loss = _fit(wn.reshape(B, M), sel)
    return (loss.reshape(()), lin)
